# baseline (speedup 1.0000x reference)
import functools

import jax
import jax.numpy as jnp
from jax import lax
from jax.experimental import pallas as pl
from jax.experimental.pallas import tpu as pltpu
from jax.experimental.pallas import tpu_sc as plsc

_NC = 2
_NS = 16
_NW = _NC * _NS
_LANES = 16


def _scores_tc(table, W1, b1, w2):
    V, D = table.shape
    BLK = 2000
    NB = V // BLK

    def body(t_ref, w1_ref, b1_ref, w2_ref, o_ref):
        h = jnp.tanh(
            jnp.dot(t_ref[...], w1_ref[...], preferred_element_type=jnp.float32)
            + b1_ref[...][None, :]
        )
        o_ref[0, 0, :] = jnp.sum(h * w2_ref[...][None, :], axis=1)

    s3 = pl.pallas_call(
        body,
        grid=(NB,),
        in_specs=[
            pl.BlockSpec((BLK, D), lambda i: (i, 0)),
            pl.BlockSpec((D, D), lambda i: (0, 0)),
            pl.BlockSpec((D,), lambda i: (0,)),
            pl.BlockSpec((D,), lambda i: (0,)),
        ],
        out_specs=pl.BlockSpec((1, 1, BLK), lambda i: (i, 0, 0)),
        out_shape=jax.ShapeDtypeStruct((NB, 1, BLK), jnp.float32),
    )(table, W1, b1, w2)
    return s3.reshape(V)


def _gather_logits_sc(s, xf):
    (V,) = s.shape
    (N,) = xf.shape
    per_w = N // _NW
    CH = 6400
    NCH = per_w // CH
    mesh = plsc.VectorSubcoreMesh(core_axis_name="c", subcore_axis_name="s")

    @functools.partial(
        pl.kernel,
        out_type=jax.ShapeDtypeStruct((N,), jnp.float32),
        mesh=mesh,
        scratch_types=[
            pltpu.VMEM((V,), jnp.float32),
            pltpu.VMEM((2, CH), jnp.int32),
            pltpu.VMEM((2, CH), jnp.float32),
            pltpu.SemaphoreType.DMA,
            pltpu.SemaphoreType.DMA,
            pltpu.SemaphoreType.DMA,
            pltpu.SemaphoreType.DMA,
            pltpu.SemaphoreType.DMA,
        ],
        compiler_params=pltpu.CompilerParams(needs_layout_passes=False),
    )
    def k(s_hbm, x_hbm, out_hbm, s_v, xi_v, lo_v, ssem, sin0, sin1, sout0,
          sout1):
        wid = lax.axis_index("s") * _NC + lax.axis_index("c")
        base = wid * per_w
        sins = (sin0, sin1)
        souts = (sout0, sout1)

        def in_desc(c):
            return pltpu.make_async_copy(
                x_hbm.at[pl.ds(base + c * CH, CH)], xi_v.at[c % 2], sins[c % 2]
            )

        def out_desc(c):
            return pltpu.make_async_copy(
                lo_v.at[c % 2], out_hbm.at[pl.ds(base + c * CH, CH)],
                souts[c % 2]
            )

        scp = pltpu.make_async_copy(s_hbm, s_v, ssem)
        scp.start()
        in_desc(0).start()
        scp.wait()
        for c in range(NCH):
            in_desc(c).wait()
            if c + 1 < NCH:
                in_desc(c + 1).start()
            if c >= 2:
                out_desc(c - 2).wait()

            def vec(i, _):
                for u in range(4):
                    off = (i * 4 + u) * _LANES
                    idx = xi_v[c % 2, pl.ds(off, _LANES)]
                    lo_v[c % 2, pl.ds(off, _LANES)] = plsc.load_gather(
                        s_v, [idx]
                    )
                return 0

            lax.fori_loop(0, CH // (4 * _LANES), vec, 0)
            out_desc(c).start()
        out_desc(NCH - 2).wait()
        out_desc(NCH - 1).wait()

    return k(s, xf)


def _pool_sc(xf, lof, table, B, L):
    V, D = table.shape
    RPW = B // _NW
    NVEC = D // _LANES
    C0, C1 = 104, 96
    NFULL = L // _LANES
    TAIL = L - NFULL * _LANES
    NEG_MASK = jnp.float32(-1e9)
    NEG_PAD = jnp.float32(-3e38)
    mesh = plsc.VectorSubcoreMesh(core_axis_name="c", subcore_axis_name="s")

    @functools.partial(
        pl.kernel,
        out_type=jax.ShapeDtypeStruct((B, D), jnp.float32),
        mesh=mesh,
        scratch_types=[
            pltpu.VMEM((RPW * L + _LANES,), jnp.int32),
            pltpu.VMEM((RPW * L + _LANES,), jnp.float32),
            pltpu.VMEM(((NFULL + 1) * _LANES,), jnp.float32),
            pltpu.VMEM((L, D), jnp.float32),
            pltpu.VMEM((L, D), jnp.float32),
            pltpu.VMEM((RPW, D), jnp.float32),
            pltpu.SemaphoreType.DMA,
            pltpu.SemaphoreType.DMA,
        ],
        compiler_params=pltpu.CompilerParams(needs_layout_passes=False),
    )
    def k(x_hbm, lo_hbm, t_hbm, out_hbm, xi_v, lo_v, al_v, rows0, rows1,
          out_v, sem0, sem1):
        wid = lax.axis_index("s") * _NC + lax.axis_index("c")
        rbase = wid * RPW
        pltpu.sync_copy(x_hbm.at[pl.ds(rbase * L, RPW * L)],
                        xi_v.at[pl.ds(0, RPW * L)])
        pltpu.sync_copy(lo_hbm.at[pl.ds(rbase * L, RPW * L)],
                        lo_v.at[pl.ds(0, RPW * L)])
        bufs = ((rows0, sem0), (rows1, sem1))

        def softmax_row(r):
            base = r * L
            pos0 = lax.broadcasted_iota(jnp.int32, (_LANES,), 0)
            cnt = jnp.int32(0)
            for c in range(NFULL + 1):
                xv = xi_v[pl.ds(base + c * _LANES, _LANES)]
                valid = xv != 0
                if c == NFULL:
                    valid = valid & (pos0 + c * _LANES < L)
                cnt = cnt + jnp.sum(jnp.where(valid, 1, 0).astype(jnp.int32))
            m = NEG_PAD
            lms = []
            for c in range(NFULL + 1):
                lv = lo_v[pl.ds(base + c * _LANES, _LANES)]
                pos = pos0 + c * _LANES
                lm = jnp.where(pos < cnt, lv, NEG_MASK)
                if c == NFULL:
                    lm = jnp.where(pos < L, lm, NEG_PAD)
                lms.append(lm)
                m = jnp.maximum(m, jnp.max(lm))
            z = jnp.float32(0.0)
            for c, lm in enumerate(lms):
                e = jnp.exp(lm - m)
                z = z + jnp.sum(e)
                al_v[pl.ds(c * _LANES, _LANES)] = e
            return z

        def gather_descs(r, rows_b, sem_b):
            return (
                pltpu.make_async_copy(
                    t_hbm.at[xi_v.at[pl.ds(r * L, C0)]],
                    rows_b.at[pl.ds(0, C0)],
                    sem_b,
                ),
                pltpu.make_async_copy(
                    t_hbm.at[xi_v.at[pl.ds(r * L + C0, C1)]],
                    rows_b.at[pl.ds(C0, C1)],
                    sem_b,
                ),
            )

        def issue(r, rows_b, sem_b):
            for cp in gather_descs(r, rows_b, sem_b):
                cp.start()

        def drain(r, rows_b, sem_b):
            for cp in gather_descs(r, rows_b, sem_b):
                cp.wait()

        def fma16(rows_b, a16, lbase, acc, nlanes):
            for c in range(nlanes):
                a = a16[c]
                acc = tuple(
                    acc[j] + a * rows_b[lbase + c, pl.ds(j * _LANES, _LANES)]
                    for j in range(NVEC)
                )
            return acc

        issue(0, rows0, sem0)
        issue(1, rows1, sem1)

        def pair(g, _):
            for b, (rows_b, sem_b) in enumerate(bufs):
                r = g * 2 + b
                z = softmax_row(r)
                drain(r, rows_b, sem_b)

                def lstep(li, acc):
                    a16 = al_v[pl.ds(li * _LANES, _LANES)]
                    return fma16(rows_b, a16, li * _LANES, acc, _LANES)

                acc0 = tuple(
                    jnp.zeros((_LANES,), jnp.float32) for _ in range(NVEC)
                )
                acc = lax.fori_loop(0, NFULL, lstep, acc0, unroll=2)
                a16 = al_v[pl.ds(NFULL * _LANES, _LANES)]
                acc = fma16(rows_b, a16, NFULL * _LANES, acc, TAIL)
                rz = jnp.ones((_LANES,), jnp.float32) / jnp.broadcast_to(
                    z, (_LANES,)
                )
                for j in range(NVEC):
                    out_v[r, pl.ds(j * _LANES, _LANES)] = acc[j] * rz

                @pl.when(r + 2 < RPW)
                def _():
                    issue(r + 2, rows_b, sem_b)

            return 0

        lax.fori_loop(0, RPW // 2, pair, 0)
        pltpu.sync_copy(out_v, out_hbm.at[pl.ds(rbase, RPW)])

    return k(xf, lof, table)


def kernel(x, table, W1, b1, w2):
    B, L = x.shape
    xf = x.reshape(B * L).astype(jnp.int32)
    s = _scores_tc(table, W1, b1, w2)
    logits = _gather_logits_sc(s, xf)
    out = _pool_sc(xf, logits, table, B, L)
    return out


# speedup vs baseline: 1.0021x; 1.0021x over previous
import functools

import jax
import jax.numpy as jnp
from jax import lax
from jax.experimental import pallas as pl
from jax.experimental.pallas import tpu as pltpu
from jax.experimental.pallas import tpu_sc as plsc

_NC = 2
_NS = 16
_NW = _NC * _NS
_LANES = 16


def _scores_tc(table, W1, b1, w2):
    V, D = table.shape
    BLK = 2000
    NB = V // BLK

    def body(t_ref, w1_ref, b1_ref, w2_ref, o_ref):
        h = jnp.tanh(
            jnp.dot(t_ref[...], w1_ref[...], preferred_element_type=jnp.float32)
            + b1_ref[...][None, :]
        )
        o_ref[0, 0, :] = jnp.sum(h * w2_ref[...][None, :], axis=1)

    s3 = pl.pallas_call(
        body,
        grid=(NB,),
        in_specs=[
            pl.BlockSpec((BLK, D), lambda i: (i, 0)),
            pl.BlockSpec((D, D), lambda i: (0, 0)),
            pl.BlockSpec((D,), lambda i: (0,)),
            pl.BlockSpec((D,), lambda i: (0,)),
        ],
        out_specs=pl.BlockSpec((1, 1, BLK), lambda i: (i, 0, 0)),
        out_shape=jax.ShapeDtypeStruct((NB, 1, BLK), jnp.float32),
    )(table, W1, b1, w2)
    return s3.reshape(V)


def _gather_logits_sc(s, xf):
    (V,) = s.shape
    (N,) = xf.shape
    per_w = N // _NW
    CH = 6400
    NCH = per_w // CH
    mesh = plsc.VectorSubcoreMesh(core_axis_name="c", subcore_axis_name="s")

    @functools.partial(
        pl.kernel,
        out_type=jax.ShapeDtypeStruct((N,), jnp.float32),
        mesh=mesh,
        scratch_types=[
            pltpu.VMEM((V,), jnp.float32),
            pltpu.VMEM((CH,), jnp.int32),
            pltpu.VMEM((CH,), jnp.float32),
        ],
        compiler_params=pltpu.CompilerParams(needs_layout_passes=False),
    )
    def k(s_hbm, x_hbm, out_hbm, s_v, xi_v, lo_v):
        wid = lax.axis_index("s") * _NC + lax.axis_index("c")
        base = wid * per_w
        pltpu.sync_copy(s_hbm, s_v)

        def chunk(ci, _):
            off = base + ci * CH
            pltpu.sync_copy(x_hbm.at[pl.ds(off, CH)], xi_v)

            def vec(i, _):
                for u in range(4):
                    voff = (i * 4 + u) * _LANES
                    idx = xi_v[pl.ds(voff, _LANES)]
                    lo_v[pl.ds(voff, _LANES)] = plsc.load_gather(s_v, [idx])
                return 0

            lax.fori_loop(0, CH // (4 * _LANES), vec, 0)
            pltpu.sync_copy(lo_v, out_hbm.at[pl.ds(off, CH)])
            return 0

        lax.fori_loop(0, NCH, chunk, 0)

    return k(s, xf)


def _pool_sc(xf, lof, table, B, L):
    V, D = table.shape
    RPW = B // _NW
    NVEC = D // _LANES
    C0, C1 = 104, 96
    NFULL = L // _LANES
    TAIL = L - NFULL * _LANES
    NEG_MASK = jnp.float32(-1e9)
    NEG_PAD = jnp.float32(-3e38)
    mesh = plsc.VectorSubcoreMesh(core_axis_name="c", subcore_axis_name="s")

    @functools.partial(
        pl.kernel,
        out_type=jax.ShapeDtypeStruct((B, D), jnp.float32),
        mesh=mesh,
        scratch_types=[
            pltpu.VMEM((RPW * L + _LANES,), jnp.int32),
            pltpu.VMEM((RPW * L + _LANES,), jnp.float32),
            pltpu.VMEM(((NFULL + 1) * _LANES,), jnp.float32),
            pltpu.VMEM((L, D), jnp.float32),
            pltpu.VMEM((L, D), jnp.float32),
            pltpu.VMEM((RPW, D), jnp.float32),
            pltpu.SemaphoreType.DMA,
            pltpu.SemaphoreType.DMA,
        ],
        compiler_params=pltpu.CompilerParams(needs_layout_passes=False),
    )
    def k(x_hbm, lo_hbm, t_hbm, out_hbm, xi_v, lo_v, al_v, rows0, rows1,
          out_v, sem0, sem1):
        wid = lax.axis_index("s") * _NC + lax.axis_index("c")
        rbase = wid * RPW
        pltpu.sync_copy(x_hbm.at[pl.ds(rbase * L, RPW * L)],
                        xi_v.at[pl.ds(0, RPW * L)])
        pltpu.sync_copy(lo_hbm.at[pl.ds(rbase * L, RPW * L)],
                        lo_v.at[pl.ds(0, RPW * L)])
        bufs = ((rows0, sem0), (rows1, sem1))

        def softmax_row(r):
            base = r * L
            pos0 = lax.broadcasted_iota(jnp.int32, (_LANES,), 0)
            cnt = jnp.int32(0)
            for c in range(NFULL + 1):
                xv = xi_v[pl.ds(base + c * _LANES, _LANES)]
                valid = xv != 0
                if c == NFULL:
                    valid = valid & (pos0 + c * _LANES < L)
                cnt = cnt + jnp.sum(jnp.where(valid, 1, 0).astype(jnp.int32))
            m = NEG_PAD
            lms = []
            for c in range(NFULL + 1):
                lv = lo_v[pl.ds(base + c * _LANES, _LANES)]
                pos = pos0 + c * _LANES
                lm = jnp.where(pos < cnt, lv, NEG_MASK)
                if c == NFULL:
                    lm = jnp.where(pos < L, lm, NEG_PAD)
                lms.append(lm)
                m = jnp.maximum(m, jnp.max(lm))
            z = jnp.float32(0.0)
            for c, lm in enumerate(lms):
                e = jnp.exp(lm - m)
                z = z + jnp.sum(e)
                al_v[pl.ds(c * _LANES, _LANES)] = e
            return z

        def gather_descs(r, rows_b, sem_b):
            return (
                pltpu.make_async_copy(
                    t_hbm.at[xi_v.at[pl.ds(r * L, C0)]],
                    rows_b.at[pl.ds(0, C0)],
                    sem_b,
                ),
                pltpu.make_async_copy(
                    t_hbm.at[xi_v.at[pl.ds(r * L + C0, C1)]],
                    rows_b.at[pl.ds(C0, C1)],
                    sem_b,
                ),
            )

        def issue(r, rows_b, sem_b):
            for cp in gather_descs(r, rows_b, sem_b):
                cp.start()

        def drain(r, rows_b, sem_b):
            for cp in gather_descs(r, rows_b, sem_b):
                cp.wait()

        def fma16(rows_b, a16, lbase, acc, nlanes):
            for c in range(nlanes):
                a = a16[c]
                acc = tuple(
                    acc[j] + a * rows_b[lbase + c, pl.ds(j * _LANES, _LANES)]
                    for j in range(NVEC)
                )
            return acc

        issue(0, rows0, sem0)
        issue(1, rows1, sem1)

        def pair(g, _):
            for b, (rows_b, sem_b) in enumerate(bufs):
                r = g * 2 + b
                z = softmax_row(r)
                drain(r, rows_b, sem_b)

                def lstep(li, acc):
                    a16 = al_v[pl.ds(li * _LANES, _LANES)]
                    return fma16(rows_b, a16, li * _LANES, acc, _LANES)

                acc0 = tuple(
                    jnp.zeros((_LANES,), jnp.float32) for _ in range(NVEC)
                )
                acc = lax.fori_loop(0, NFULL, lstep, acc0, unroll=2)
                a16 = al_v[pl.ds(NFULL * _LANES, _LANES)]
                acc = fma16(rows_b, a16, NFULL * _LANES, acc, TAIL)
                rz = jnp.ones((_LANES,), jnp.float32) / jnp.broadcast_to(
                    z, (_LANES,)
                )
                for j in range(NVEC):
                    out_v[r, pl.ds(j * _LANES, _LANES)] = acc[j] * rz

                @pl.when(r + 2 < RPW)
                def _():
                    issue(r + 2, rows_b, sem_b)

            return 0

        lax.fori_loop(0, RPW // 2, pair, 0)
        pltpu.sync_copy(out_v, out_hbm.at[pl.ds(rbase, RPW)])

    return k(xf, lof, table)


def kernel(x, table, W1, b1, w2):
    B, L = x.shape
    xf = x.reshape(B * L).astype(jnp.int32)
    s = _scores_tc(table, W1, b1, w2)
    logits = _gather_logits_sc(s, xf)
    out = _pool_sc(xf, logits, table, B, L)
    return out
